# initial kernel scaffold (unmeasured)
import jax
import jax.numpy as jnp
from jax import lax
from jax.experimental import pallas as pl
from jax.experimental.pallas import tpu as pltpu

N_DEV = 32
LOG2_N = 5
B, Sq, D = 2, 256, 768
Hq, Dh = 8, 64
HD = Hq * Dh
BH = B * Hq


def kernel(x, Wq, Wo, K_ext, V_ext):
    skv_loc = K_ext.shape[1]

    def body(x_ref, wq_ref, wo_ref, k_ref, v_ref, out_ref,
             o_acc, ml_acc, o_rx, ml_rx,
             o_send_sems, o_recv_sems, ml_send_sems, ml_recv_sems):
        my = lax.axis_index("i")

        x2 = x_ref[...].reshape(B * Sq, D).astype(jnp.bfloat16)
        wq = wq_ref[...].astype(jnp.bfloat16)
        qT = lax.dot_general(wq, x2, (((0,), (1,)), ((), ())),
                             preferred_element_type=jnp.float32)
        qT = qT * 0.125

        for b in range(B):
            kb = k_ref[b, :, :, :].reshape(skv_loc, HD).astype(jnp.bfloat16)
            vb = v_ref[b, :, :, :].reshape(skv_loc, HD).astype(jnp.bfloat16)
            for h in range(Hq):
                bh = b * Hq + h
                q_bh = qT[h * Dh:(h + 1) * Dh,
                          b * Sq:(b + 1) * Sq].astype(jnp.bfloat16)
                k_bh = kb[:, h * Dh:(h + 1) * Dh]
                v_bh = vb[:, h * Dh:(h + 1) * Dh]
                sT = lax.dot_general(k_bh, q_bh, (((1,), (0,)), ((), ())),
                                     preferred_element_type=jnp.float32)
                m = jnp.max(sT, axis=0, keepdims=True)
                p = jnp.exp(sT - m)
                l = jnp.sum(p, axis=0, keepdims=True)
                oT = lax.dot_general(v_bh, p.astype(jnp.bfloat16),
                                     (((0,), (0,)), ((), ())),
                                     preferred_element_type=jnp.float32)
                o_acc[bh, :, :] = oT
                ml_acc[bh, 0:1, :] = m
                ml_acc[bh, 1:2, :] = l

        for step in range(LOG2_N):
            partner = my ^ (1 << step)
            o_rd = pltpu.make_async_remote_copy(
                src_ref=o_acc,
                dst_ref=o_rx.at[step],
                send_sem=o_send_sems.at[step],
                recv_sem=o_recv_sems.at[step],
                device_id=(partner,),
                device_id_type=pl.DeviceIdType.MESH,
            )
            ml_rd = pltpu.make_async_remote_copy(
                src_ref=ml_acc,
                dst_ref=ml_rx.at[step],
                send_sem=ml_send_sems.at[step],
                recv_sem=ml_recv_sems.at[step],
                device_id=(partner,),
                device_id_type=pl.DeviceIdType.MESH,
            )
            o_rd.start()
            ml_rd.start()
            o_rd.wait()
            ml_rd.wait()

            m1 = ml_acc[:, 0:1, :]
            l1 = ml_acc[:, 1:2, :]
            m2 = ml_rx[step, :, 0:1, :]
            l2 = ml_rx[step, :, 1:2, :]
            mn = jnp.maximum(m1, m2)
            a1 = jnp.exp(m1 - mn)
            a2 = jnp.exp(m2 - mn)
            ml_acc[:, 0:1, :] = mn
            ml_acc[:, 1:2, :] = a1 * l1 + a2 * l2
            o_acc[...] = a1 * o_acc[...] + a2 * o_rx[step, :, :, :]

        linv = 1.0 / ml_acc[:, 1:2, :]
        wo = wo_ref[...].astype(jnp.bfloat16)
        for b in range(B):
            acc = jnp.zeros((Sq, D), jnp.float32)
            for h in range(Hq):
                bh = b * Hq + h
                o_n = (o_acc[bh, :, :] * linv[bh, :, :]).astype(jnp.bfloat16)
                wo_h = wo[h * Dh:(h + 1) * Dh, :]
                acc = acc + lax.dot_general(
                    o_n, wo_h, (((0,), (0,)), ((), ())),
                    preferred_element_type=jnp.float32)
            out_ref[b, :, :] = acc

    return pl.pallas_call(
        body,
        out_shape=jax.ShapeDtypeStruct((B, Sq, D), jnp.float32),
        in_specs=[pl.BlockSpec(memory_space=pltpu.VMEM)] * 5,
        out_specs=pl.BlockSpec(memory_space=pltpu.VMEM),
        scratch_shapes=[
            pltpu.VMEM((BH, Dh, Sq), jnp.float32),
            pltpu.VMEM((BH, 2, Sq), jnp.float32),
            pltpu.VMEM((LOG2_N, BH, Dh, Sq), jnp.float32),
            pltpu.VMEM((LOG2_N, BH, 2, Sq), jnp.float32),
            pltpu.SemaphoreType.DMA((LOG2_N,)),
            pltpu.SemaphoreType.DMA((LOG2_N,)),
            pltpu.SemaphoreType.DMA((LOG2_N,)),
            pltpu.SemaphoreType.DMA((LOG2_N,)),
        ],
        compiler_params=pltpu.CompilerParams(collective_id=0),
    )(x, Wq, Wo, K_ext, V_ext)


# baseline (device time: 122233 ns/iter reference)
import jax
import jax.numpy as jnp
from jax import lax
from jax.experimental import pallas as pl
from jax.experimental.pallas import tpu as pltpu

N_DEV = 32
LOG2_N = 5
B, Sq, D = 2, 256, 768
Hq, Dh = 8, 64
HD = Hq * Dh
BH = B * Hq


def kernel(x, Wq, Wo, K_ext, V_ext):
    skv_loc = K_ext.shape[1]

    def body(x_ref, wq_ref, wo_ref, k_ref, v_ref, out_ref,
             o_acc, ml_acc, o_rx, ml_rx,
             o_send_sems, o_recv_sems, ml_send_sems, ml_recv_sems):
        my = lax.axis_index("i")

        x2 = x_ref[...].reshape(B * Sq, D).astype(jnp.bfloat16)
        wq = wq_ref[...].astype(jnp.bfloat16)
        qT = lax.dot_general(wq, x2, (((0,), (1,)), ((), ())),
                             preferred_element_type=jnp.float32)
        qT = qT * 0.125

        for b in range(B):
            kb = k_ref[b, :, :, :].reshape(skv_loc, HD).astype(jnp.bfloat16)
            vb = v_ref[b, :, :, :].reshape(skv_loc, HD).astype(jnp.bfloat16)
            for h in range(Hq):
                bh = b * Hq + h
                q_bh = qT[h * Dh:(h + 1) * Dh,
                          b * Sq:(b + 1) * Sq].astype(jnp.bfloat16)
                k_bh = kb[:, h * Dh:(h + 1) * Dh]
                v_bh = vb[:, h * Dh:(h + 1) * Dh]
                sT = lax.dot_general(k_bh, q_bh, (((1,), (0,)), ((), ())),
                                     preferred_element_type=jnp.float32)
                m = jnp.max(sT, axis=0, keepdims=True)
                p = jnp.exp(sT - m)
                l = jnp.sum(p, axis=0, keepdims=True)
                oT = lax.dot_general(v_bh, p.astype(jnp.bfloat16),
                                     (((0,), (0,)), ((), ())),
                                     preferred_element_type=jnp.float32)
                o_acc[bh, :, :] = oT
                ml_acc[bh, 0:1, :] = m
                ml_acc[bh, 1:2, :] = l

        for step in range(LOG2_N):
            partner = my ^ (1 << step)
            o_rd = pltpu.make_async_remote_copy(
                src_ref=o_acc,
                dst_ref=o_rx.at[step],
                send_sem=o_send_sems.at[step],
                recv_sem=o_recv_sems.at[step],
                device_id=(partner,),
                device_id_type=pl.DeviceIdType.MESH,
            )
            ml_rd = pltpu.make_async_remote_copy(
                src_ref=ml_acc,
                dst_ref=ml_rx.at[step],
                send_sem=ml_send_sems.at[step],
                recv_sem=ml_recv_sems.at[step],
                device_id=(partner,),
                device_id_type=pl.DeviceIdType.MESH,
            )
            o_rd.start()
            ml_rd.start()
            o_rd.wait()
            ml_rd.wait()

            m1 = ml_acc[:, 0:1, :]
            l1 = ml_acc[:, 1:2, :]
            m2 = ml_rx[step, :, 0:1, :]
            l2 = ml_rx[step, :, 1:2, :]
            mn = jnp.maximum(m1, m2)
            a1 = jnp.exp(m1 - mn)
            a2 = jnp.exp(m2 - mn)
            ml_acc[:, 0:1, :] = mn
            ml_acc[:, 1:2, :] = a1 * l1 + a2 * l2
            o_acc[...] = a1 * o_acc[...] + a2 * o_rx[step, :, :, :]

        linv = 1.0 / ml_acc[:, 1:2, :]
        wo = wo_ref[...].astype(jnp.bfloat16)
        for b in range(B):
            acc = jnp.zeros((Sq, D), jnp.float32)
            for h in range(Hq):
                bh = b * Hq + h
                o_n = (o_acc[bh, :, :] * linv[bh, :, :]).astype(jnp.bfloat16)
                wo_h = wo[h * Dh:(h + 1) * Dh, :]
                acc = acc + lax.dot_general(
                    o_n, wo_h, (((0,), (0,)), ((), ())),
                    preferred_element_type=jnp.float32)
            out_ref[b, :, :] = acc

    return pl.pallas_call(
        body,
        out_shape=jax.ShapeDtypeStruct((B, Sq, D), jnp.float32),
        in_specs=[pl.BlockSpec(memory_space=pltpu.VMEM)] * 5,
        out_specs=pl.BlockSpec(memory_space=pltpu.VMEM),
        scratch_shapes=[
            pltpu.VMEM((BH, Dh, Sq), jnp.float32),
            pltpu.VMEM((BH, 2, Sq), jnp.float32),
            pltpu.VMEM((LOG2_N, BH, Dh, Sq), jnp.float32),
            pltpu.VMEM((LOG2_N, BH, 2, Sq), jnp.float32),
            pltpu.SemaphoreType.DMA((LOG2_N,)),
            pltpu.SemaphoreType.DMA((LOG2_N,)),
            pltpu.SemaphoreType.DMA((LOG2_N,)),
            pltpu.SemaphoreType.DMA((LOG2_N,)),
        ],
    )(x, Wq, Wo, K_ext, V_ext)


# device time: 83261 ns/iter; 1.4681x vs baseline; 1.4681x over previous
import jax
import jax.numpy as jnp
from jax import lax
from jax.experimental import pallas as pl
from jax.experimental.pallas import tpu as pltpu

N_DEV = 32
LOG2_N = 5
B, Sq, D = 2, 256, 768
Hq, Dh = 8, 64
HD = Hq * Dh
BH = B * Hq


def kernel(x, Wq, Wo, K_ext, V_ext):
    skv_loc = K_ext.shape[1]

    def body(x_ref, wq_ref, wo_ref, k_ref, v_ref, out_ref,
             o_acc, ml_acc, o_rx, ml_rx,
             o_send_sems, o_recv_sems, ml_send_sems, ml_recv_sems):
        my = lax.axis_index("i")

        x2 = x_ref[...].reshape(B * Sq, D).astype(jnp.bfloat16)
        wq = wq_ref[...].astype(jnp.bfloat16)
        qT = lax.dot_general(wq, x2, (((0,), (1,)), ((), ())),
                             preferred_element_type=jnp.float32)
        qT = qT * 0.125

        for b in range(B):
            kb = k_ref[b, :, :, :].reshape(skv_loc, HD).astype(jnp.bfloat16)
            vb = v_ref[b, :, :, :].reshape(skv_loc, HD).astype(jnp.bfloat16)
            for h in range(Hq):
                bh = b * Hq + h
                q_bh = qT[h * Dh:(h + 1) * Dh,
                          b * Sq:(b + 1) * Sq].astype(jnp.bfloat16)
                k_bh = kb[:, h * Dh:(h + 1) * Dh]
                v_bh = vb[:, h * Dh:(h + 1) * Dh]
                sT = lax.dot_general(k_bh, q_bh, (((1,), (0,)), ((), ())),
                                     preferred_element_type=jnp.float32)
                m = jnp.max(sT, axis=0, keepdims=True)
                p = jnp.exp(sT - m)
                l = jnp.sum(p, axis=0, keepdims=True)
                oT = lax.dot_general(v_bh, p.astype(jnp.bfloat16),
                                     (((0,), (0,)), ((), ())),
                                     preferred_element_type=jnp.float32)
                o_acc[bh, :, :] = oT.astype(jnp.bfloat16)
                ml_acc[bh, 0:1, :] = m
                ml_acc[bh, 1:2, :] = l

        for step in range(LOG2_N):
            partner = my ^ (1 << step)
            o_rd = pltpu.make_async_remote_copy(
                src_ref=o_acc,
                dst_ref=o_rx.at[step],
                send_sem=o_send_sems.at[step],
                recv_sem=o_recv_sems.at[step],
                device_id=(partner,),
                device_id_type=pl.DeviceIdType.MESH,
            )
            ml_rd = pltpu.make_async_remote_copy(
                src_ref=ml_acc,
                dst_ref=ml_rx.at[step],
                send_sem=ml_send_sems.at[step],
                recv_sem=ml_recv_sems.at[step],
                device_id=(partner,),
                device_id_type=pl.DeviceIdType.MESH,
            )
            o_rd.start()
            ml_rd.start()
            o_rd.wait()
            ml_rd.wait()

            m1 = ml_acc[:, 0:1, :]
            l1 = ml_acc[:, 1:2, :]
            m2 = ml_rx[step, :, 0:1, :]
            l2 = ml_rx[step, :, 1:2, :]
            mn = jnp.maximum(m1, m2)
            a1 = jnp.exp(m1 - mn)
            a2 = jnp.exp(m2 - mn)
            ml_acc[:, 0:1, :] = mn
            ml_acc[:, 1:2, :] = a1 * l1 + a2 * l2
            o_new = (a1 * o_acc[...].astype(jnp.float32)
                     + a2 * o_rx[step, :, :, :].astype(jnp.float32))
            o_acc[...] = o_new.astype(jnp.bfloat16)

        linv = 1.0 / ml_acc[:, 1:2, :]
        wo = wo_ref[...].astype(jnp.bfloat16)
        for b in range(B):
            acc = jnp.zeros((Sq, D), jnp.float32)
            for h in range(Hq):
                bh = b * Hq + h
                o_n = (o_acc[bh, :, :].astype(jnp.float32)
                       * linv[bh, :, :]).astype(jnp.bfloat16)
                wo_h = wo[h * Dh:(h + 1) * Dh, :]
                acc = acc + lax.dot_general(
                    o_n, wo_h, (((0,), (0,)), ((), ())),
                    preferred_element_type=jnp.float32)
            out_ref[b, :, :] = acc

    return pl.pallas_call(
        body,
        out_shape=jax.ShapeDtypeStruct((B, Sq, D), jnp.float32),
        in_specs=[pl.BlockSpec(memory_space=pltpu.VMEM)] * 5,
        out_specs=pl.BlockSpec(memory_space=pltpu.VMEM),
        scratch_shapes=[
            pltpu.VMEM((BH, Dh, Sq), jnp.bfloat16),
            pltpu.VMEM((BH, 2, Sq), jnp.float32),
            pltpu.VMEM((LOG2_N, BH, Dh, Sq), jnp.bfloat16),
            pltpu.VMEM((LOG2_N, BH, 2, Sq), jnp.float32),
            pltpu.SemaphoreType.DMA((LOG2_N,)),
            pltpu.SemaphoreType.DMA((LOG2_N,)),
            pltpu.SemaphoreType.DMA((LOG2_N,)),
            pltpu.SemaphoreType.DMA((LOG2_N,)),
        ],
    )(x, Wq, Wo, K_ext, V_ext)


# device time: 64766 ns/iter; 1.8873x vs baseline; 1.2856x over previous
import jax
import jax.numpy as jnp
from jax import lax
from jax.experimental import pallas as pl
from jax.experimental.pallas import tpu as pltpu

N_DEV = 32
LOG2_N = 5
B, Sq, D = 2, 256, 768
Hq, Dh = 8, 64
HD = Hq * Dh
BH = B * Hq
SH = Sq // 2

BITS_A = (0, 1, 2, 3, 4)
BITS_B = (2, 3, 4, 0, 1)


def kernel(x, Wq, Wo, K_ext, V_ext):
    skv_loc = K_ext.shape[1]

    def body(x_ref, wq_ref, wo_ref, k_ref, v_ref, out_ref,
             o_acc, ml_acc, o_rx, ml_rx,
             o_send_sems, o_recv_sems, ml_send_sems, ml_recv_sems):
        my = lax.axis_index("i")

        x2 = x_ref[...].reshape(B * Sq, D).astype(jnp.bfloat16)
        wq = wq_ref[...].astype(jnp.bfloat16)
        qT = lax.dot_general(wq, x2, (((0,), (1,)), ((), ())),
                             preferred_element_type=jnp.float32)
        qT = qT * 0.125

        kbs = []
        vbs = []
        for b in range(B):
            kbs.append(k_ref[b, :, :, :].reshape(skv_loc, HD).astype(jnp.bfloat16))
            vbs.append(v_ref[b, :, :, :].reshape(skv_loc, HD).astype(jnp.bfloat16))

        def flash_half(half):
            for b in range(B):
                for h in range(Hq):
                    bh = b * Hq + h
                    c0 = b * Sq + half * SH
                    q_bh = qT[h * Dh:(h + 1) * Dh,
                              c0:c0 + SH].astype(jnp.bfloat16)
                    k_bh = kbs[b][:, h * Dh:(h + 1) * Dh]
                    v_bh = vbs[b][:, h * Dh:(h + 1) * Dh]
                    sT = lax.dot_general(k_bh, q_bh, (((1,), (0,)), ((), ())),
                                         preferred_element_type=jnp.float32)
                    m = jnp.max(sT, axis=0, keepdims=True)
                    p = jnp.exp(sT - m)
                    l = jnp.sum(p, axis=0, keepdims=True)
                    oT = lax.dot_general(v_bh, p.astype(jnp.bfloat16),
                                         (((0,), (0,)), ((), ())),
                                         preferred_element_type=jnp.float32)
                    o_acc[half, bh, :, :] = oT.astype(jnp.bfloat16)
                    ml_acc[half, bh, 0:1, :] = m
                    ml_acc[half, bh, 1:2, :] = l

        def start_exchange(half, step):
            bit = (BITS_A, BITS_B)[half][step]
            partner = my ^ (1 << bit)
            o_rd = pltpu.make_async_remote_copy(
                src_ref=o_acc.at[half],
                dst_ref=o_rx.at[step, half],
                send_sem=o_send_sems.at[step, half],
                recv_sem=o_recv_sems.at[step, half],
                device_id=(partner,),
                device_id_type=pl.DeviceIdType.MESH,
            )
            ml_rd = pltpu.make_async_remote_copy(
                src_ref=ml_acc.at[half],
                dst_ref=ml_rx.at[step, half],
                send_sem=ml_send_sems.at[step, half],
                recv_sem=ml_recv_sems.at[step, half],
                device_id=(partner,),
                device_id_type=pl.DeviceIdType.MESH,
            )
            o_rd.start()
            ml_rd.start()
            return o_rd, ml_rd

        def combine(half, step):
            m1 = ml_acc[half, :, 0:1, :]
            l1 = ml_acc[half, :, 1:2, :]
            m2 = ml_rx[step, half, :, 0:1, :]
            l2 = ml_rx[step, half, :, 1:2, :]
            mn = jnp.maximum(m1, m2)
            a1 = jnp.exp(m1 - mn)
            a2 = jnp.exp(m2 - mn)
            ml_acc[half, :, 0:1, :] = mn
            ml_acc[half, :, 1:2, :] = a1 * l1 + a2 * l2
            o_new = (a1 * o_acc[half].astype(jnp.float32)
                     + a2 * o_rx[step, half].astype(jnp.float32))
            o_acc[half] = o_new.astype(jnp.bfloat16)

        flash_half(0)
        rd0 = start_exchange(0, 0)
        flash_half(1)
        rd1 = start_exchange(1, 0)
        for step in range(LOG2_N):
            rd0[0].wait()
            rd0[1].wait()
            combine(0, step)
            if step + 1 < LOG2_N:
                rd0 = start_exchange(0, step + 1)
            rd1[0].wait()
            rd1[1].wait()
            combine(1, step)
            if step + 1 < LOG2_N:
                rd1 = start_exchange(1, step + 1)

        wo = wo_ref[...].astype(jnp.bfloat16)
        for b in range(B):
            for half in range(2):
                linv = 1.0 / ml_acc[half, :, 1:2, :]
                acc = jnp.zeros((SH, D), jnp.float32)
                for h in range(Hq):
                    bh = b * Hq + h
                    o_n = (o_acc[half, bh, :, :].astype(jnp.float32)
                           * linv[bh, :, :]).astype(jnp.bfloat16)
                    wo_h = wo[h * Dh:(h + 1) * Dh, :]
                    acc = acc + lax.dot_general(
                        o_n, wo_h, (((0,), (0,)), ((), ())),
                        preferred_element_type=jnp.float32)
                out_ref[b, half * SH:(half + 1) * SH, :] = acc

    return pl.pallas_call(
        body,
        out_shape=jax.ShapeDtypeStruct((B, Sq, D), jnp.float32),
        in_specs=[pl.BlockSpec(memory_space=pltpu.VMEM)] * 5,
        out_specs=pl.BlockSpec(memory_space=pltpu.VMEM),
        scratch_shapes=[
            pltpu.VMEM((2, BH, Dh, SH), jnp.bfloat16),
            pltpu.VMEM((2, BH, 2, SH), jnp.float32),
            pltpu.VMEM((LOG2_N, 2, BH, Dh, SH), jnp.bfloat16),
            pltpu.VMEM((LOG2_N, 2, BH, 2, SH), jnp.float32),
            pltpu.SemaphoreType.DMA((LOG2_N, 2)),
            pltpu.SemaphoreType.DMA((LOG2_N, 2)),
            pltpu.SemaphoreType.DMA((LOG2_N, 2)),
            pltpu.SemaphoreType.DMA((LOG2_N, 2)),
        ],
    )(x, Wq, Wo, K_ext, V_ext)
